# baseline (speedup 1.0000x reference)
import jax
import jax.numpy as jnp
from jax.experimental import pallas as pl

N_NODES = 10000
K = 9
SETTING = [(1, 16, 1, 1), (6, 24, 2, 2), (6, 32, 3, 2), (6, 64, 4, 2), (6, 96, 3, 1)]
STEM = 32


def _bn(x, g, b, eps=1e-5):
    m = jnp.mean(x, axis=0, keepdims=True)
    v = jnp.var(x, axis=0, keepdims=True)
    return g * (x - m) / jnp.sqrt(v + eps) + b


def _relu6(x):
    return jnp.clip(x, 0.0, 6.0)


def _identity_pallas(x):
    def body(x_ref, o_ref):
        o_ref[...] = x_ref[...]
    return pl.pallas_call(
        body, out_shape=jax.ShapeDtypeStruct(x.shape, x.dtype))(x)


def kernel(x, params, edge_index, kernel_offsets):
    x = _identity_pallas(x)
    src = edge_index[0]
    dst = edge_index[1]
    n = x.shape[0]
    msgs = jnp.einsum('ei,eio->eo', x[src], params['Wc1'][kernel_offsets])
    h = jax.ops.segment_sum(msgs, dst, num_segments=n)
    h = _relu6(_bn(h, params['g0'], params['b0']))
    in_c = STEM
    bi = 0
    for t, c, nrep, s in SETTING:
        for i in range(nrep):
            stride = s if i == 0 else 1
            p = params['blocks'][bi]
            bi += 1
            use_res = (in_c == c and stride == 1)
            ident = h
            z = _relu6(_bn(h @ p['W1'], p['g1'], p['b1']))
            m = z[src] * p['Wdw'][kernel_offsets]
            z = jax.ops.segment_sum(m, dst, num_segments=n)
            z = _relu6(_bn(z, p['g2'], p['b2']))
            z = _bn(z @ p['W3'], p['g3'], p['b3'])
            h = z + ident if use_res else z
            in_c = c
    h = _relu6(_bn(h @ params['Wc8'], params['g8'], params['b8']))
    pooled = jnp.mean(h, axis=0, keepdims=True)
    out = pooled @ params['Wfc'] + params['bfc']
    return out


# speedup vs baseline: 3.1145x; 3.1145x over previous
import functools

import jax
import jax.numpy as jnp
from jax import lax
from jax.experimental import pallas as pl
from jax.experimental.pallas import tpu as pltpu
from jax.experimental.pallas import tpu_sc as plsc

N = 10000
NPAD = 10112
ROWS_PER_TILE = 632
E = 90000
NT = 16
EPT = 5632
NCHUNK = 44
CB = 128
K = 9
R = 8
STEM = 32
SETTING = [(1, 16, 1, 1), (6, 24, 2, 2), (6, 32, 3, 2), (6, 64, 4, 2), (6, 96, 3, 1)]
_CG = {32: 16, 96: 48, 144: 72, 192: 96, 384: 96, 576: 96}


def _bn(x, g, b, eps=1e-5):
    m = jnp.mean(x, axis=0, keepdims=True)
    v = jnp.var(x, axis=0, keepdims=True)
    return g * (x - m) / jnp.sqrt(v + eps) + b


def _relu6(x):
    return jnp.clip(x, 0.0, 6.0)


@functools.cache
def _dw_kernel(Cg, G):
    G2 = G // 2
    mesh = plsc.VectorSubcoreMesh(core_axis_name="c", subcore_axis_name="s")
    CSL = Cg // 16

    def body(z_hbm, w_hbm, sidx_hbm, widx_hbm, dstm_hbm, out_hbm,
             sidx_v, widx_v, dst_v, rows_v, wrows_v, acc_sh, sem_g, sem_w):
        c = lax.axis_index("c")
        s = lax.axis_index("s")
        pltpu.sync_copy(dstm_hbm.at[s], dst_v)
        for p in range(G2):
            g = c * G2 + p
            pltpu.sync_copy(sidx_hbm.at[g, s], sidx_v)
            pltpu.sync_copy(widx_hbm.at[g, s], widx_v)

            @pl.loop(0, CB)
            def _zero(e):
                for csl in range(CSL):
                    rows_v[e, pl.ds(csl * 16, 16)] = jnp.zeros((16,), jnp.float32)

            for rep in range(4):
                pltpu.sync_copy(rows_v,
                                acc_sh.at[pl.ds(s * ROWS_PER_TILE + rep * CB, CB)])
            pltpu.sync_copy(rows_v.at[pl.ds(0, ROWS_PER_TILE - 4 * CB)],
                            acc_sh.at[pl.ds(s * ROWS_PER_TILE + 4 * CB,
                                            ROWS_PER_TILE - 4 * CB)])
            plsc.subcore_barrier()

            @pl.loop(0, NCHUNK)
            def _chunk(j):
                cp1 = pltpu.async_copy(z_hbm.at[sidx_v.at[j]], rows_v, sem_g)
                cp2 = pltpu.async_copy(w_hbm.at[widx_v.at[j]], wrows_v, sem_w)
                cp1.wait()
                cp2.wait()

                @pl.loop(0, CB)
                def _mul(e):
                    for csl in range(CSL):
                        sl = pl.ds(csl * 16, 16)
                        rows_v[e, sl] = rows_v[e, sl] * wrows_v[e, sl]

                pltpu.sync_copy(rows_v, acc_sh.at[dst_v.at[j]], add=True)

            plsc.subcore_barrier()
            pltpu.sync_copy(
                acc_sh.at[pl.ds(s * ROWS_PER_TILE, ROWS_PER_TILE)],
                out_hbm.at[pl.ds(g * NPAD + s * ROWS_PER_TILE, ROWS_PER_TILE)])

    return pl.kernel(
        body,
        out_type=jax.ShapeDtypeStruct((G * NPAD, Cg), jnp.float32),
        mesh=mesh,
        compiler_params=pltpu.CompilerParams(use_tc_tiling_on_sc=False),
        scratch_types=[
            pltpu.VMEM((NCHUNK, CB), jnp.int32),
            pltpu.VMEM((NCHUNK, CB), jnp.int32),
            pltpu.VMEM((NCHUNK, CB), jnp.int32),
            pltpu.VMEM((CB, Cg), jnp.float32),
            pltpu.VMEM((CB, Cg), jnp.float32),
            pltpu.VMEM_SHARED((NPAD, Cg), jnp.float32),
            pltpu.SemaphoreType.DMA,
            pltpu.SemaphoreType.DMA,
        ],
    )


def _edge_meta(edge_index, kernel_offsets):
    src = edge_index[0].astype(jnp.int32)
    dst = edge_index[1].astype(jnp.int32)
    ko = kernel_offsets.astype(jnp.int32)
    pad = NT * EPT - E
    e_ar = jnp.arange(NT * EPT, dtype=jnp.int32)
    src_p = jnp.concatenate([src, jnp.zeros((pad,), jnp.int32)])
    dst_p = jnp.concatenate([dst, N + (jnp.arange(pad, dtype=jnp.int32) % NT)])
    ko_p = jnp.concatenate([ko, jnp.zeros((pad,), jnp.int32)])
    koi = ko_p * R + (e_ar % R)
    shp = (NT, NCHUNK, CB)
    return (src_p.reshape(shp), dst_p.reshape(shp), koi.reshape(shp))


def _dw_conv(z, wdw, meta):
    hid = z.shape[1]
    Cg = _CG[hid]
    G = hid // Cg
    srcm, dstm, koim = meta
    z3 = z.reshape(N, G, Cg).transpose(1, 0, 2).reshape(G * N, Cg)
    w3 = wdw.reshape(K, G, Cg).transpose(1, 0, 2)
    wrep = jnp.broadcast_to(w3[:, :, None, :], (G, K, R, Cg)).reshape(G * K * R, Cg)
    goff = jnp.arange(G, dtype=jnp.int32)[:, None, None, None]
    sidx = srcm[None] + goff * N
    widx = koim[None] + goff * (K * R)
    out = _dw_kernel(Cg, G)(z3, wrep, sidx, widx, dstm)
    out3 = out.reshape(G, NPAD, Cg)[:, :N, :]
    return out3.transpose(1, 0, 2).reshape(N, hid)


def kernel(x, params, edge_index, kernel_offsets):
    meta = _edge_meta(edge_index, kernel_offsets)
    src = edge_index[0]
    dst = edge_index[1]
    n = x.shape[0]
    msgs = jnp.einsum('ei,eio->eo', x[src], params['Wc1'][kernel_offsets])
    h = jax.ops.segment_sum(msgs, dst, num_segments=n)
    h = _relu6(_bn(h, params['g0'], params['b0']))
    in_c = STEM
    bi = 0
    for t, c, nrep, s in SETTING:
        for i in range(nrep):
            stride = s if i == 0 else 1
            p = params['blocks'][bi]
            bi += 1
            use_res = (in_c == c and stride == 1)
            ident = h
            z = _relu6(_bn(h @ p['W1'], p['g1'], p['b1']))
            z = _dw_conv(z, p['Wdw'], meta)
            z = _relu6(_bn(z, p['g2'], p['b2']))
            z = _bn(z @ p['W3'], p['g3'], p['b3'])
            h = z + ident if use_res else z
            in_c = c
    h = _relu6(_bn(h @ params['Wc8'], params['g8'], params['b8']))
    pooled = jnp.mean(h, axis=0, keepdims=True)
    out = pooled @ params['Wfc'] + params['bfc']
    return out


# speedup vs baseline: 4.1317x; 1.3266x over previous
import functools

import jax
import jax.numpy as jnp
from jax import lax
from jax.experimental import pallas as pl
from jax.experimental.pallas import tpu as pltpu
from jax.experimental.pallas import tpu_sc as plsc

N = 10000
NPAD = 10112
ROWS_PER_TILE = 632
E = 90000
NT = 16
EPT = 5632
NCHUNK = 44
CB = 128
K = 9
R = 8
STEM = 32
SETTING = [(1, 16, 1, 1), (6, 24, 2, 2), (6, 32, 3, 2), (6, 64, 4, 2), (6, 96, 3, 1)]
_CG = {32: 16, 64: 32, 96: 48, 144: 72, 192: 96, 384: 96, 576: 96}


def _bn(x, g, b, eps=1e-5):
    m = jnp.mean(x, axis=0, keepdims=True)
    v = jnp.var(x, axis=0, keepdims=True)
    return g * (x - m) / jnp.sqrt(v + eps) + b


def _relu6(x):
    return jnp.clip(x, 0.0, 6.0)


@functools.cache
def _dw_kernel(Cg, G):
    G2 = G // 2
    mesh = plsc.VectorSubcoreMesh(core_axis_name="c", subcore_axis_name="s")
    CSL = Cg // 16

    def body(z_hbm, w_hbm, sidx_hbm, widx_hbm, dstm_hbm, out_hbm,
             sidx_v, widx_v, dst_v, rows_a, wrows_a, rows_b, wrows_b,
             acc_sh, sem_ga, sem_wa, sem_gb, sem_wb):
        c = lax.axis_index("c")
        s = lax.axis_index("s")
        pltpu.sync_copy(dstm_hbm.at[s], dst_v)

        def start_gather(j, rows, wrows, sg, sw):
            pltpu.async_copy(z_hbm.at[sidx_v.at[j]], rows, sg)
            pltpu.async_copy(w_hbm.at[widx_v.at[j]], wrows, sw)

        def wait_gather(j, rows, wrows, sg, sw):
            pltpu.make_async_copy(z_hbm.at[sidx_v.at[j]], rows, sg).wait()
            pltpu.make_async_copy(w_hbm.at[widx_v.at[j]], wrows, sw).wait()

        def mul(rows, wrows):
            @plsc.parallel_loop(0, CB, unroll=4)
            def _mul(e):
                for csl in range(CSL):
                    sl = pl.ds(csl * 16, 16)
                    rows[e, sl] = rows[e, sl] * wrows[e, sl]

        for p in range(G2):
            g = c * G2 + p
            pltpu.sync_copy(sidx_hbm.at[g, s], sidx_v)
            pltpu.sync_copy(widx_hbm.at[g, s], widx_v)

            @plsc.parallel_loop(0, CB, unroll=4)
            def _zero(e):
                for csl in range(CSL):
                    rows_a[e, pl.ds(csl * 16, 16)] = jnp.zeros((16,), jnp.float32)

            for rep in range(4):
                pltpu.sync_copy(rows_a,
                                acc_sh.at[pl.ds(s * ROWS_PER_TILE + rep * CB, CB)])
            pltpu.sync_copy(rows_a.at[pl.ds(0, ROWS_PER_TILE - 4 * CB)],
                            acc_sh.at[pl.ds(s * ROWS_PER_TILE + 4 * CB,
                                            ROWS_PER_TILE - 4 * CB)])
            plsc.subcore_barrier()

            start_gather(0, rows_a, wrows_a, sem_ga, sem_wa)

            @pl.loop(0, NCHUNK // 2)
            def _pair(jj):
                j = jj * 2
                start_gather(j + 1, rows_b, wrows_b, sem_gb, sem_wb)
                wait_gather(j, rows_a, wrows_a, sem_ga, sem_wa)
                mul(rows_a, wrows_a)
                pltpu.sync_copy(rows_a, acc_sh.at[dst_v.at[j]], add=True)

                @pl.when(j + 2 < NCHUNK)
                def _():
                    start_gather(j + 2, rows_a, wrows_a, sem_ga, sem_wa)

                wait_gather(j + 1, rows_b, wrows_b, sem_gb, sem_wb)
                mul(rows_b, wrows_b)
                pltpu.sync_copy(rows_b, acc_sh.at[dst_v.at[j + 1]], add=True)

            plsc.subcore_barrier()
            pltpu.sync_copy(
                acc_sh.at[pl.ds(s * ROWS_PER_TILE, ROWS_PER_TILE)],
                out_hbm.at[pl.ds(g * NPAD + s * ROWS_PER_TILE, ROWS_PER_TILE)])

    return pl.kernel(
        body,
        out_type=jax.ShapeDtypeStruct((G * NPAD, Cg), jnp.float32),
        mesh=mesh,
        compiler_params=pltpu.CompilerParams(use_tc_tiling_on_sc=False),
        scratch_types=[
            pltpu.VMEM((NCHUNK, CB), jnp.int32),
            pltpu.VMEM((NCHUNK, CB), jnp.int32),
            pltpu.VMEM((NCHUNK, CB), jnp.int32),
            pltpu.VMEM((CB, Cg), jnp.float32),
            pltpu.VMEM((CB, Cg), jnp.float32),
            pltpu.VMEM((CB, Cg), jnp.float32),
            pltpu.VMEM((CB, Cg), jnp.float32),
            pltpu.VMEM_SHARED((NPAD, Cg), jnp.float32),
            pltpu.SemaphoreType.DMA,
            pltpu.SemaphoreType.DMA,
            pltpu.SemaphoreType.DMA,
            pltpu.SemaphoreType.DMA,
        ],
    )


def _edge_meta(edge_index, kernel_offsets):
    src = edge_index[0].astype(jnp.int32)
    dst = edge_index[1].astype(jnp.int32)
    ko = kernel_offsets.astype(jnp.int32)
    pad = NT * EPT - E
    e_ar = jnp.arange(NT * EPT, dtype=jnp.int32)
    src_p = jnp.concatenate([src, jnp.zeros((pad,), jnp.int32)])
    dst_p = jnp.concatenate([dst, N + (jnp.arange(pad, dtype=jnp.int32) % NT)])
    ko_p = jnp.concatenate([ko, jnp.zeros((pad,), jnp.int32)])
    koi = ko_p * R + (e_ar % R)
    shp = (NT, NCHUNK, CB)
    return (src_p.reshape(shp), dst_p.reshape(shp), koi.reshape(shp))


def _dw_conv(z, wdw, meta):
    hid = z.shape[1]
    Cg = _CG[hid]
    G = hid // Cg
    srcm, dstm, koim = meta
    z3 = z.reshape(N, G, Cg).transpose(1, 0, 2).reshape(G * N, Cg)
    w3 = wdw.reshape(K, G, Cg).transpose(1, 0, 2)
    wrep = jnp.broadcast_to(w3[:, :, None, :], (G, K, R, Cg)).reshape(G * K * R, Cg)
    goff = jnp.arange(G, dtype=jnp.int32)[:, None, None, None]
    sidx = srcm[None] + goff * N
    widx = koim[None] + goff * (K * R)
    out = _dw_kernel(Cg, G)(z3, wrep, sidx, widx, dstm)
    out3 = out.reshape(G, NPAD, Cg)[:, :N, :]
    return out3.transpose(1, 0, 2).reshape(N, hid)


def kernel(x, params, edge_index, kernel_offsets):
    meta = _edge_meta(edge_index, kernel_offsets)
    zstem = jnp.concatenate(
        [jnp.broadcast_to(x[:, 0:1], (N, STEM)),
         jnp.broadcast_to(x[:, 1:2], (N, STEM))], axis=1)
    wstem = jnp.concatenate(
        [params['Wc1'][:, 0, :], params['Wc1'][:, 1, :]], axis=1)
    hs = _dw_conv(zstem, wstem, meta)
    h = hs[:, :STEM] + hs[:, STEM:]
    h = _relu6(_bn(h, params['g0'], params['b0']))
    in_c = STEM
    bi = 0
    for t, c, nrep, s in SETTING:
        for i in range(nrep):
            stride = s if i == 0 else 1
            p = params['blocks'][bi]
            bi += 1
            use_res = (in_c == c and stride == 1)
            ident = h
            z = _relu6(_bn(h @ p['W1'], p['g1'], p['b1']))
            z = _dw_conv(z, p['Wdw'], meta)
            z = _relu6(_bn(z, p['g2'], p['b2']))
            z = _bn(z @ p['W3'], p['g3'], p['b3'])
            h = z + ident if use_res else z
            in_c = c
    h = _relu6(_bn(h @ params['Wc8'], params['g8'], params['b8']))
    pooled = jnp.mean(h, axis=0, keepdims=True)
    out = pooled @ params['Wfc'] + params['bfc']
    return out


# speedup vs baseline: 4.7772x; 1.1563x over previous
import functools

import jax
import jax.numpy as jnp
from jax import lax
from jax.experimental import pallas as pl
from jax.experimental.pallas import tpu as pltpu
from jax.experimental.pallas import tpu_sc as plsc

N = 10000
NPAD = 10112
ROWS_PER_TILE = 632
E = 90000
NT = 16
EPT = 5632
NCHUNK = 44
CB = 128
K = 9
R = 8
STEM = 32
SETTING = [(1, 16, 1, 1), (6, 24, 2, 2), (6, 32, 3, 2), (6, 64, 4, 2), (6, 96, 3, 1)]
_CG = {32: 16, 64: 32, 96: 48, 144: 72, 192: 96, 384: 96, 576: 96}


def _bn(x, g, b, eps=1e-5):
    m = jnp.mean(x, axis=0, keepdims=True)
    v = jnp.var(x, axis=0, keepdims=True)
    return g * (x - m) / jnp.sqrt(v + eps) + b


def _relu6(x):
    return jnp.clip(x, 0.0, 6.0)


@functools.cache
def _dw_kernel(Cg, G):
    G2 = G // 2
    mesh = plsc.VectorSubcoreMesh(core_axis_name="c", subcore_axis_name="s")
    CSL = Cg // 16

    def body(z_hbm, w_hbm, sidx_hbm, kom_hbm, dstm_hbm, out_hbm,
             sidx_v, ko_v, dst_v, wtab_v, rows_a, rows_b,
             acc_sh, sem_ga, sem_gb):
        c = lax.axis_index("c")
        s = lax.axis_index("s")
        pltpu.sync_copy(dstm_hbm.at[s], dst_v)
        pltpu.sync_copy(kom_hbm.at[s], ko_v)

        def start_gather(j, rows, sg):
            pltpu.async_copy(z_hbm.at[sidx_v.at[j]], rows, sg)

        def wait_gather(j, rows, sg):
            pltpu.make_async_copy(z_hbm.at[sidx_v.at[j]], rows, sg).wait()

        def mul(j, rows):
            @plsc.parallel_loop(0, CB // 16)
            def _mul(eb):
                e0 = eb * 16
                kos = ko_v[j, pl.ds(e0, 16)]
                for lane in range(16):
                    ko = kos[lane]
                    for csl in range(CSL):
                        sl = pl.ds(csl * 16, 16)
                        rows[e0 + lane, sl] = (rows[e0 + lane, sl]
                                               * wtab_v[ko, sl])

        for p in range(G2):
            g = c * G2 + p
            pltpu.sync_copy(sidx_hbm.at[g, s], sidx_v)
            pltpu.sync_copy(w_hbm.at[g], wtab_v)

            @plsc.parallel_loop(0, CB, unroll=4)
            def _zero(e):
                for csl in range(CSL):
                    rows_a[e, pl.ds(csl * 16, 16)] = jnp.zeros((16,), jnp.float32)

            for rep in range(4):
                pltpu.sync_copy(rows_a,
                                acc_sh.at[pl.ds(s * ROWS_PER_TILE + rep * CB, CB)])
            pltpu.sync_copy(rows_a.at[pl.ds(0, ROWS_PER_TILE - 4 * CB)],
                            acc_sh.at[pl.ds(s * ROWS_PER_TILE + 4 * CB,
                                            ROWS_PER_TILE - 4 * CB)])
            plsc.subcore_barrier()

            start_gather(0, rows_a, sem_ga)

            @pl.loop(0, NCHUNK // 2)
            def _pair(jj):
                j = jj * 2
                start_gather(j + 1, rows_b, sem_gb)
                wait_gather(j, rows_a, sem_ga)
                mul(j, rows_a)
                pltpu.sync_copy(rows_a, acc_sh.at[dst_v.at[j]], add=True)

                @pl.when(j + 2 < NCHUNK)
                def _():
                    start_gather(j + 2, rows_a, sem_ga)

                wait_gather(j + 1, rows_b, sem_gb)
                mul(j + 1, rows_b)
                pltpu.sync_copy(rows_b, acc_sh.at[dst_v.at[j + 1]], add=True)

            plsc.subcore_barrier()
            pltpu.sync_copy(
                acc_sh.at[pl.ds(s * ROWS_PER_TILE, ROWS_PER_TILE)],
                out_hbm.at[pl.ds(g * NPAD + s * ROWS_PER_TILE, ROWS_PER_TILE)])

    return pl.kernel(
        body,
        out_type=jax.ShapeDtypeStruct((G * NPAD, Cg), jnp.float32),
        mesh=mesh,
        compiler_params=pltpu.CompilerParams(use_tc_tiling_on_sc=False),
        scratch_types=[
            pltpu.VMEM((NCHUNK, CB), jnp.int32),
            pltpu.VMEM((NCHUNK, CB), jnp.int32),
            pltpu.VMEM((NCHUNK, CB), jnp.int32),
            pltpu.VMEM((K, Cg), jnp.float32),
            pltpu.VMEM((CB, Cg), jnp.float32),
            pltpu.VMEM((CB, Cg), jnp.float32),
            pltpu.VMEM_SHARED((NPAD, Cg), jnp.float32),
            pltpu.SemaphoreType.DMA,
            pltpu.SemaphoreType.DMA,
        ],
    )


def _edge_meta(edge_index, kernel_offsets):
    src = edge_index[0].astype(jnp.int32)
    dst = edge_index[1].astype(jnp.int32)
    ko = kernel_offsets.astype(jnp.int32)
    pad = NT * EPT - E
    e_ar = jnp.arange(NT * EPT, dtype=jnp.int32)
    src_p = jnp.concatenate([src, jnp.zeros((pad,), jnp.int32)])
    dst_p = jnp.concatenate([dst, N + (jnp.arange(pad, dtype=jnp.int32) % NT)])
    ko_p = jnp.concatenate([ko, jnp.zeros((pad,), jnp.int32)])
    shp = (NT, NCHUNK, CB)
    return (src_p.reshape(shp), dst_p.reshape(shp), ko_p.reshape(shp))


def _dw_conv(z, wdw, meta):
    hid = z.shape[1]
    Cg = _CG[hid]
    G = hid // Cg
    srcm, dstm, kom = meta
    z3 = z.reshape(N, G, Cg).transpose(1, 0, 2).reshape(G * N, Cg)
    w3 = wdw.reshape(K, G, Cg).transpose(1, 0, 2)
    goff = jnp.arange(G, dtype=jnp.int32)[:, None, None, None]
    sidx = srcm[None] + goff * N
    out = _dw_kernel(Cg, G)(z3, w3, sidx, kom, dstm)
    out3 = out.reshape(G, NPAD, Cg)[:, :N, :]
    return out3.transpose(1, 0, 2).reshape(N, hid)


def kernel(x, params, edge_index, kernel_offsets):
    meta = _edge_meta(edge_index, kernel_offsets)
    zstem = jnp.concatenate(
        [jnp.broadcast_to(x[:, 0:1], (N, STEM)),
         jnp.broadcast_to(x[:, 1:2], (N, STEM))], axis=1)
    wstem = jnp.concatenate(
        [params['Wc1'][:, 0, :], params['Wc1'][:, 1, :]], axis=1)
    hs = _dw_conv(zstem, wstem, meta)
    h = hs[:, :STEM] + hs[:, STEM:]
    h = _relu6(_bn(h, params['g0'], params['b0']))
    in_c = STEM
    bi = 0
    for t, c, nrep, s in SETTING:
        for i in range(nrep):
            stride = s if i == 0 else 1
            p = params['blocks'][bi]
            bi += 1
            use_res = (in_c == c and stride == 1)
            ident = h
            z = _relu6(_bn(h @ p['W1'], p['g1'], p['b1']))
            z = _dw_conv(z, p['Wdw'], meta)
            z = _relu6(_bn(z, p['g2'], p['b2']))
            z = _bn(z @ p['W3'], p['g3'], p['b3'])
            h = z + ident if use_res else z
            in_c = c
    h = _relu6(_bn(h @ params['Wc8'], params['g8'], params['b8']))
    pooled = jnp.mean(h, axis=0, keepdims=True)
    out = pooled @ params['Wfc'] + params['bfc']
    return out


# speedup vs baseline: 5.4792x; 1.1469x over previous
import functools

import jax
import jax.numpy as jnp
from jax import lax
from jax.experimental import pallas as pl
from jax.experimental.pallas import tpu as pltpu
from jax.experimental.pallas import tpu_sc as plsc

N = 10000
NPAD = 10112
ROWS_PER_TILE = 632
BN_BLK = 632
NB = 16
E = 90000
NT = 16
EPT = 5632
NCHUNK = 44
CB = 128
K = 9
STEM = 32
EPS = 1e-5
SETTING = [(1, 16, 1, 1), (6, 24, 2, 2), (6, 32, 3, 2), (6, 64, 4, 2), (6, 96, 3, 1)]
_CG = {32: 16, 64: 32, 96: 48, 144: 72, 192: 96, 384: 96, 576: 96}


@functools.cache
def _dw_kernel(Cg, G):
    G2 = G // 2
    mesh = plsc.VectorSubcoreMesh(core_axis_name="c", subcore_axis_name="s")
    CSL = Cg // 16

    def body(z_hbm, w_hbm, sidx_hbm, kom_hbm, dstm_hbm, out_hbm,
             sidx_v, ko_v, dst_v, wtab_v, rows_a, rows_b,
             acc_sh, sem_ga, sem_gb):
        c = lax.axis_index("c")
        s = lax.axis_index("s")
        pltpu.sync_copy(dstm_hbm.at[s], dst_v)
        pltpu.sync_copy(kom_hbm.at[s], ko_v)

        def start_gather(j, rows, sg):
            pltpu.async_copy(z_hbm.at[sidx_v.at[j]], rows, sg)

        def wait_gather(j, rows, sg):
            pltpu.make_async_copy(z_hbm.at[sidx_v.at[j]], rows, sg).wait()

        def mul(j, rows):
            @plsc.parallel_loop(0, CB // 16)
            def _mul(eb):
                e0 = eb * 16
                kos = ko_v[j, pl.ds(e0, 16)]
                for lane in range(16):
                    ko = kos[lane]
                    for csl in range(CSL):
                        sl = pl.ds(csl * 16, 16)
                        rows[e0 + lane, sl] = (rows[e0 + lane, sl]
                                               * wtab_v[ko, sl])

        @pl.loop(0, G2)
        def _group(p):
            g = c * G2 + p
            pltpu.sync_copy(sidx_hbm.at[g, s], sidx_v)
            pltpu.sync_copy(w_hbm.at[g], wtab_v)

            @plsc.parallel_loop(0, CB, unroll=4)
            def _zero(e):
                for csl in range(CSL):
                    rows_a[e, pl.ds(csl * 16, 16)] = jnp.zeros((16,), jnp.float32)

            for rep in range(4):
                pltpu.sync_copy(rows_a,
                                acc_sh.at[pl.ds(s * ROWS_PER_TILE + rep * CB, CB)])
            pltpu.sync_copy(rows_a.at[pl.ds(0, ROWS_PER_TILE - 4 * CB)],
                            acc_sh.at[pl.ds(s * ROWS_PER_TILE + 4 * CB,
                                            ROWS_PER_TILE - 4 * CB)])
            plsc.subcore_barrier()

            start_gather(0, rows_a, sem_ga)

            @pl.loop(0, NCHUNK // 2)
            def _pair(jj):
                j = jj * 2
                start_gather(j + 1, rows_b, sem_gb)
                wait_gather(j, rows_a, sem_ga)
                mul(j, rows_a)
                pltpu.sync_copy(rows_a, acc_sh.at[dst_v.at[j]], add=True)

                @pl.when(j + 2 < NCHUNK)
                def _():
                    start_gather(j + 2, rows_a, sem_ga)

                wait_gather(j + 1, rows_b, sem_gb)
                mul(j + 1, rows_b)
                pltpu.sync_copy(rows_b, acc_sh.at[dst_v.at[j + 1]], add=True)

            plsc.subcore_barrier()
            pltpu.sync_copy(
                acc_sh.at[pl.ds(s * ROWS_PER_TILE, ROWS_PER_TILE)],
                out_hbm.at[pl.ds(g * NPAD + s * ROWS_PER_TILE, ROWS_PER_TILE)])

    return pl.kernel(
        body,
        out_type=jax.ShapeDtypeStruct((G * NPAD, Cg), jnp.float32),
        mesh=mesh,
        compiler_params=pltpu.CompilerParams(use_tc_tiling_on_sc=False),
        scratch_types=[
            pltpu.VMEM((NCHUNK, CB), jnp.int32),
            pltpu.VMEM((NCHUNK, CB), jnp.int32),
            pltpu.VMEM((NCHUNK, CB), jnp.int32),
            pltpu.VMEM((K, Cg), jnp.float32),
            pltpu.VMEM((CB, Cg), jnp.float32),
            pltpu.VMEM((CB, Cg), jnp.float32),
            pltpu.VMEM_SHARED((NPAD, Cg), jnp.float32),
            pltpu.SemaphoreType.DMA,
            pltpu.SemaphoreType.DMA,
        ],
    )


def _edge_meta(edge_index, kernel_offsets):
    src = edge_index[0].astype(jnp.int32)
    dst = edge_index[1].astype(jnp.int32)
    ko = kernel_offsets.astype(jnp.int32)
    pad = NT * EPT - E
    src_p = jnp.concatenate([src, jnp.zeros((pad,), jnp.int32)])
    dst_p = jnp.concatenate([dst, N + (jnp.arange(pad, dtype=jnp.int32) % NT)])
    ko_p = jnp.concatenate([ko, jnp.zeros((pad,), jnp.int32)])
    shp = (NT, NCHUNK, CB)
    return (src_p.reshape(shp), dst_p.reshape(shp), ko_p.reshape(shp))


def _dw_conv(z3, wdw, meta):
    hid = wdw.shape[1]
    Cg = _CG[hid]
    G = hid // Cg
    srcm, dstm, kom = meta
    w3 = wdw.reshape(K, G, Cg).transpose(1, 0, 2)
    goff = jnp.arange(G, dtype=jnp.int32)[:, None, None, None]
    sidx = srcm[None] + goff * NPAD
    return _dw_kernel(Cg, G)(z3, w3, sidx, kom, dstm)


def _row_mask(nb, x):
    rows = lax.broadcasted_iota(jnp.int32, x.shape, 0) + nb * BN_BLK
    return jnp.where(rows < N, x, 0.0)


def _ab(stats_ref, g, b):
    m = stats_ref[0:1, :] * (1.0 / N)
    v = stats_ref[1:2, :] * (1.0 / N) - m * m
    a = g * lax.rsqrt(v + EPS)
    return a, b - a * m


@functools.cache
def _mm_bn_kernel(in_c, hid, Cg):
    G = hid // Cg

    def body(h_ref, w_ref, g_ref, b_ref, o_ref, zbuf, stats):
        p = pl.program_id(0)
        nb = pl.program_id(1)

        @pl.when(jnp.logical_and(p == 0, nb == 0))
        def _():
            stats[...] = jnp.zeros_like(stats)

        @pl.when(p == 0)
        def _():
            z = jnp.dot(h_ref[...], w_ref[...],
                        preferred_element_type=jnp.float32)
            zbuf[pl.ds(nb * BN_BLK, BN_BLK), :] = z
            zm = _row_mask(nb, z)
            stats[0:1, :] += jnp.sum(zm, axis=0, keepdims=True)
            stats[1:2, :] += jnp.sum(zm * zm, axis=0, keepdims=True)

        @pl.when(p == 1)
        def _():
            a, bb = _ab(stats, g_ref[...], b_ref[...])
            zn = jnp.clip(a * zbuf[pl.ds(nb * BN_BLK, BN_BLK), :] + bb, 0.0, 6.0)
            for g in range(G):
                o_ref[g] = zn[:, g * Cg:(g + 1) * Cg]

    return pl.pallas_call(
        body,
        grid=(2, NB),
        in_specs=[
            pl.BlockSpec((BN_BLK, in_c), lambda p, nb: (nb, 0)),
            pl.BlockSpec((in_c, hid), lambda p, nb: (0, 0)),
            pl.BlockSpec((1, hid), lambda p, nb: (0, 0)),
            pl.BlockSpec((1, hid), lambda p, nb: (0, 0)),
        ],
        out_specs=pl.BlockSpec((G, BN_BLK, Cg), lambda p, nb: (0, nb, 0)),
        out_shape=jax.ShapeDtypeStruct((G, NPAD, Cg), jnp.float32),
        scratch_shapes=[
            pltpu.VMEM((NPAD, hid), jnp.float32),
            pltpu.VMEM((2, hid), jnp.float32),
        ],
    )


@functools.cache
def _bn_mm_bn_kernel(hid, Cg, out_c, use_res):
    G = hid // Cg

    def body(z_ref, w_ref, g2_ref, b2_ref, g3_ref, b3_ref, *rest):
        if use_res:
            id_ref, o_ref, zbuf, st2, st3 = rest
        else:
            o_ref, zbuf, st2, st3 = rest
        p = pl.program_id(0)
        nb = pl.program_id(1)

        @pl.when(jnp.logical_and(p == 0, nb == 0))
        def _():
            st2[...] = jnp.zeros_like(st2)
            st3[...] = jnp.zeros_like(st3)

        def z2blk():
            return jnp.concatenate([z_ref[g] for g in range(G)], axis=1)

        @pl.when(p == 0)
        def _():
            zm = _row_mask(nb, z2blk())
            st2[0:1, :] += jnp.sum(zm, axis=0, keepdims=True)
            st2[1:2, :] += jnp.sum(zm * zm, axis=0, keepdims=True)

        @pl.when(p == 1)
        def _():
            a, bb = _ab(st2, g2_ref[...], b2_ref[...])
            zn = jnp.clip(a * z2blk() + bb, 0.0, 6.0)
            z3 = jnp.dot(zn, w_ref[...], preferred_element_type=jnp.float32)
            zbuf[pl.ds(nb * BN_BLK, BN_BLK), :] = z3
            zm = _row_mask(nb, z3)
            st3[0:1, :] += jnp.sum(zm, axis=0, keepdims=True)
            st3[1:2, :] += jnp.sum(zm * zm, axis=0, keepdims=True)

        @pl.when(p == 2)
        def _():
            a, bb = _ab(st3, g3_ref[...], b3_ref[...])
            h = a * zbuf[pl.ds(nb * BN_BLK, BN_BLK), :] + bb
            if use_res:
                h = h + id_ref[...]
            o_ref[...] = _row_mask(nb, h)

    in_specs = [
        pl.BlockSpec((G, BN_BLK, Cg), lambda p, nb: (0, nb, 0)),
        pl.BlockSpec((hid, out_c), lambda p, nb: (0, 0)),
        pl.BlockSpec((1, hid), lambda p, nb: (0, 0)),
        pl.BlockSpec((1, hid), lambda p, nb: (0, 0)),
        pl.BlockSpec((1, out_c), lambda p, nb: (0, 0)),
        pl.BlockSpec((1, out_c), lambda p, nb: (0, 0)),
    ]
    if use_res:
        in_specs.append(pl.BlockSpec((BN_BLK, out_c), lambda p, nb: (nb, 0)))
    return pl.pallas_call(
        body,
        grid=(3, NB),
        in_specs=in_specs,
        out_specs=pl.BlockSpec((BN_BLK, out_c), lambda p, nb: (nb, 0)),
        out_shape=jax.ShapeDtypeStruct((NPAD, out_c), jnp.float32),
        scratch_shapes=[
            pltpu.VMEM((NPAD, out_c), jnp.float32),
            pltpu.VMEM((2, hid), jnp.float32),
            pltpu.VMEM((2, out_c), jnp.float32),
        ],
    )


@functools.cache
def _stem_bn_kernel():
    def body(z_ref, g_ref, b_ref, o_ref, zbuf, stats):
        p = pl.program_id(0)
        nb = pl.program_id(1)

        @pl.when(jnp.logical_and(p == 0, nb == 0))
        def _():
            stats[...] = jnp.zeros_like(stats)

        @pl.when(p == 0)
        def _():
            z = z_ref[0] + z_ref[1]
            zbuf[pl.ds(nb * BN_BLK, BN_BLK), :] = z
            zm = _row_mask(nb, z)
            stats[0:1, :] += jnp.sum(zm, axis=0, keepdims=True)
            stats[1:2, :] += jnp.sum(zm * zm, axis=0, keepdims=True)

        @pl.when(p == 1)
        def _():
            a, bb = _ab(stats, g_ref[...], b_ref[...])
            zn = jnp.clip(a * zbuf[pl.ds(nb * BN_BLK, BN_BLK), :] + bb, 0.0, 6.0)
            o_ref[...] = _row_mask(nb, zn)

    return pl.pallas_call(
        body,
        grid=(2, NB),
        in_specs=[
            pl.BlockSpec((2, BN_BLK, STEM), lambda p, nb: (0, nb, 0)),
            pl.BlockSpec((1, STEM), lambda p, nb: (0, 0)),
            pl.BlockSpec((1, STEM), lambda p, nb: (0, 0)),
        ],
        out_specs=pl.BlockSpec((BN_BLK, STEM), lambda p, nb: (nb, 0)),
        out_shape=jax.ShapeDtypeStruct((NPAD, STEM), jnp.float32),
        scratch_shapes=[
            pltpu.VMEM((NPAD, STEM), jnp.float32),
            pltpu.VMEM((2, STEM), jnp.float32),
        ],
    )


@functools.cache
def _head_kernel(in_c, fin, ncls):
    def body(h_ref, w8_ref, g_ref, b_ref, wfc_ref, bfc_ref, o_ref,
             stats, psum):
        p = pl.program_id(0)
        nb = pl.program_id(1)

        @pl.when(jnp.logical_and(p == 0, nb == 0))
        def _():
            stats[...] = jnp.zeros_like(stats)
            psum[...] = jnp.zeros_like(psum)

        @pl.when(p == 0)
        def _():
            z = jnp.dot(h_ref[...], w8_ref[...],
                        preferred_element_type=jnp.float32)
            zm = _row_mask(nb, z)
            stats[0:1, :] += jnp.sum(zm, axis=0, keepdims=True)
            stats[1:2, :] += jnp.sum(zm * zm, axis=0, keepdims=True)

        @pl.when(p == 1)
        def _():
            z = jnp.dot(h_ref[...], w8_ref[...],
                        preferred_element_type=jnp.float32)
            a, bb = _ab(stats, g_ref[...], b_ref[...])
            zn = jnp.clip(a * z + bb, 0.0, 6.0)
            psum[...] += jnp.sum(_row_mask(nb, zn), axis=0, keepdims=True)

        @pl.when(jnp.logical_and(p == 2, nb == 0))
        def _():
            pooled = psum[...] * (1.0 / N)
            o_ref[...] = (jnp.dot(pooled, wfc_ref[...],
                                  preferred_element_type=jnp.float32)
                          + bfc_ref[...])

    return pl.pallas_call(
        body,
        grid=(3, NB),
        in_specs=[
            pl.BlockSpec((BN_BLK, in_c), lambda p, nb: (nb, 0)),
            pl.BlockSpec((in_c, fin), lambda p, nb: (0, 0)),
            pl.BlockSpec((1, fin), lambda p, nb: (0, 0)),
            pl.BlockSpec((1, fin), lambda p, nb: (0, 0)),
            pl.BlockSpec((fin, ncls), lambda p, nb: (0, 0)),
            pl.BlockSpec((1, ncls), lambda p, nb: (0, 0)),
        ],
        out_specs=pl.BlockSpec((1, ncls), lambda p, nb: (0, 0)),
        out_shape=jax.ShapeDtypeStruct((1, ncls), jnp.float32),
        scratch_shapes=[
            pltpu.VMEM((2, fin), jnp.float32),
            pltpu.VMEM((1, fin), jnp.float32),
        ],
    )


def kernel(x, params, edge_index, kernel_offsets):
    meta = _edge_meta(edge_index, kernel_offsets)
    xpad = jnp.pad(x, ((0, NPAD - N), (0, 0)))
    z3stem = jnp.concatenate(
        [jnp.broadcast_to(xpad[:, 0:1], (NPAD, STEM)),
         jnp.broadcast_to(xpad[:, 1:2], (NPAD, STEM))], axis=0)
    wstem = jnp.concatenate(
        [params['Wc1'][:, 0, :], params['Wc1'][:, 1, :]], axis=1)
    hs = _dw_conv(z3stem, wstem, meta)
    h = _stem_bn_kernel()(hs.reshape(2, NPAD, STEM),
                          params['g0'][None], params['b0'][None])
    in_c = STEM
    bi = 0
    for t, c, nrep, s in SETTING:
        for i in range(nrep):
            stride = s if i == 0 else 1
            p = params['blocks'][bi]
            bi += 1
            use_res = (in_c == c and stride == 1)
            hid = in_c * t
            Cg = _CG[hid]
            z3 = _mm_bn_kernel(in_c, hid, Cg)(
                h, p['W1'], p['g1'][None], p['b1'][None])
            z2 = _dw_conv(z3.reshape(hid // Cg * NPAD, Cg), p['Wdw'], meta)
            args = (z2.reshape(hid // Cg, NPAD, Cg), p['W3'],
                    p['g2'][None], p['b2'][None], p['g3'][None], p['b3'][None])
            if use_res:
                args = args + (h,)
            h = _bn_mm_bn_kernel(hid, Cg, c, use_res)(*args)
            in_c = c
    out = _head_kernel(in_c, params['Wc8'].shape[1], params['bfc'].shape[0])(
        h, params['Wc8'], params['g8'][None], params['b8'][None],
        params['Wfc'], params['bfc'][None])
    return out


# speedup vs baseline: 5.6602x; 1.0330x over previous
import functools

import jax
import jax.numpy as jnp
from jax import lax
from jax.experimental import pallas as pl
from jax.experimental.pallas import tpu as pltpu
from jax.experimental.pallas import tpu_sc as plsc

N = 10000
NPAD = 10112
ROWS_PER_TILE = 632
BN_BLK = 632
NB = 16
E = 90000
NT = 16
EPT = 5632
NCHUNK = 44
CB = 128
K = 9
STEM = 32
EPS = 1e-5
SETTING = [(1, 16, 1, 1), (6, 24, 2, 2), (6, 32, 3, 2), (6, 64, 4, 2), (6, 96, 3, 1)]
_CG = {32: 16, 64: 32, 96: 48, 144: 72, 192: 96, 384: 96, 576: 96}


@functools.cache
def _dw_kernel(Cg, G):
    G2 = G // 2
    mesh = plsc.VectorSubcoreMesh(core_axis_name="c", subcore_axis_name="s")
    CSL = Cg // 16

    def body(z_hbm, w_hbm, sidx_hbm, kom_hbm, dstm_hbm, out_hbm,
             sidx_v, ko_v, dst_v, wtab_v, rows_a, rows_b, rows_c,
             acc_sh, sem_ga, sem_gb, sem_gc, sem_sa, sem_sb, sem_sc):
        c = lax.axis_index("c")
        s = lax.axis_index("s")
        pltpu.sync_copy(dstm_hbm.at[s], dst_v)
        pltpu.sync_copy(kom_hbm.at[s], ko_v)

        def start_gather(j, rows, sg):
            pltpu.async_copy(z_hbm.at[sidx_v.at[j]], rows, sg)

        def wait_gather(j, rows, sg):
            pltpu.make_async_copy(z_hbm.at[sidx_v.at[j]], rows, sg).wait()

        def start_scatter(j, rows, ss):
            pltpu.async_copy(rows, acc_sh.at[dst_v.at[j]], ss, add=True)

        def wait_scatter(j, rows, ss):
            pltpu.make_async_copy(rows, acc_sh.at[dst_v.at[j]], ss).wait()

        def mul(j, rows):
            @plsc.parallel_loop(0, CB // 16)
            def _mul(eb):
                e0 = eb * 16
                kos = ko_v[j, pl.ds(e0, 16)]
                for lane in range(16):
                    ko = kos[lane]
                    for csl in range(CSL):
                        sl = pl.ds(csl * 16, 16)
                        rows[e0 + lane, sl] = (rows[e0 + lane, sl]
                                               * wtab_v[ko, sl])

        @pl.loop(0, G2)
        def _group(p):
            g = c * G2 + p
            pltpu.sync_copy(sidx_hbm.at[g, s], sidx_v)
            pltpu.sync_copy(w_hbm.at[g], wtab_v)

            @plsc.parallel_loop(0, CB, unroll=4)
            def _zero(e):
                for csl in range(CSL):
                    rows_a[e, pl.ds(csl * 16, 16)] = jnp.zeros((16,), jnp.float32)

            for rep in range(4):
                pltpu.sync_copy(rows_a,
                                acc_sh.at[pl.ds(s * ROWS_PER_TILE + rep * CB, CB)])
            pltpu.sync_copy(rows_a.at[pl.ds(0, ROWS_PER_TILE - 4 * CB)],
                            acc_sh.at[pl.ds(s * ROWS_PER_TILE + 4 * CB,
                                            ROWS_PER_TILE - 4 * CB)])
            plsc.subcore_barrier()

            bufa = (rows_a, sem_ga, sem_sa)
            bufb = (rows_b, sem_gb, sem_sb)
            bufc = (rows_c, sem_gc, sem_sc)

            start_gather(0, rows_a, sem_ga)
            start_gather(1, rows_b, sem_gb)

            def slot(j, cur, nxt2, guard_lo):
                rx, gx, sx = cur
                rz, gz, sz = nxt2
                wait_gather(j, rx, gx)
                mul(j, rx)
                start_scatter(j, rx, sx)
                if guard_lo:
                    @pl.when(j >= 1)
                    def _():
                        wait_scatter(j - 1, rz, sz)
                else:
                    wait_scatter(j - 1, rz, sz)

                @pl.when(j + 2 < NCHUNK)
                def _():
                    start_gather(j + 2, rz, gz)

            @pl.loop(0, NCHUNK // 3)
            def _ring(jj):
                j = jj * 3
                slot(j, bufa, bufc, True)
                slot(j + 1, bufb, bufa, False)
                slot(j + 2, bufc, bufb, False)

            slot(NCHUNK - 2, bufa, bufc, False)
            slot(NCHUNK - 1, bufb, bufa, False)
            wait_scatter(NCHUNK - 1, rows_b, sem_sb)

            plsc.subcore_barrier()
            pltpu.sync_copy(
                acc_sh.at[pl.ds(s * ROWS_PER_TILE, ROWS_PER_TILE)],
                out_hbm.at[pl.ds(g * NPAD + s * ROWS_PER_TILE, ROWS_PER_TILE)])

    return pl.kernel(
        body,
        out_type=jax.ShapeDtypeStruct((G * NPAD, Cg), jnp.float32),
        mesh=mesh,
        compiler_params=pltpu.CompilerParams(use_tc_tiling_on_sc=False),
        scratch_types=[
            pltpu.VMEM((NCHUNK, CB), jnp.int32),
            pltpu.VMEM((NCHUNK, CB), jnp.int32),
            pltpu.VMEM((NCHUNK, CB), jnp.int32),
            pltpu.VMEM((K, Cg), jnp.float32),
            pltpu.VMEM((CB, Cg), jnp.float32),
            pltpu.VMEM((CB, Cg), jnp.float32),
            pltpu.VMEM((CB, Cg), jnp.float32),
            pltpu.VMEM_SHARED((NPAD, Cg), jnp.float32),
            pltpu.SemaphoreType.DMA,
            pltpu.SemaphoreType.DMA,
            pltpu.SemaphoreType.DMA,
            pltpu.SemaphoreType.DMA,
            pltpu.SemaphoreType.DMA,
            pltpu.SemaphoreType.DMA,
        ],
    )


def _edge_meta(edge_index, kernel_offsets):
    src = edge_index[0].astype(jnp.int32)
    dst = edge_index[1].astype(jnp.int32)
    ko = kernel_offsets.astype(jnp.int32)
    pad = NT * EPT - E
    src_p = jnp.concatenate([src, jnp.zeros((pad,), jnp.int32)])
    dst_p = jnp.concatenate([dst, N + (jnp.arange(pad, dtype=jnp.int32) % NT)])
    ko_p = jnp.concatenate([ko, jnp.zeros((pad,), jnp.int32)])
    shp = (NT, NCHUNK, CB)
    return (src_p.reshape(shp), dst_p.reshape(shp), ko_p.reshape(shp))


def _dw_conv(z3, wdw, meta):
    hid = wdw.shape[1]
    Cg = _CG[hid]
    G = hid // Cg
    srcm, dstm, kom = meta
    w3 = wdw.reshape(K, G, Cg).transpose(1, 0, 2)
    goff = jnp.arange(G, dtype=jnp.int32)[:, None, None, None]
    sidx = srcm[None] + goff * NPAD
    return _dw_kernel(Cg, G)(z3, w3, sidx, kom, dstm)


def _row_mask(nb, x):
    rows = lax.broadcasted_iota(jnp.int32, x.shape, 0) + nb * BN_BLK
    return jnp.where(rows < N, x, 0.0)


def _ab(stats_ref, g, b):
    m = stats_ref[0:1, :] * (1.0 / N)
    v = stats_ref[1:2, :] * (1.0 / N) - m * m
    a = g * lax.rsqrt(v + EPS)
    return a, b - a * m


@functools.cache
def _mm_bn_kernel(in_c, hid, Cg):
    G = hid // Cg

    def body(h_ref, w_ref, g_ref, b_ref, o_ref, zbuf, stats):
        p = pl.program_id(0)
        nb = pl.program_id(1)

        @pl.when(jnp.logical_and(p == 0, nb == 0))
        def _():
            stats[...] = jnp.zeros_like(stats)

        @pl.when(p == 0)
        def _():
            z = jnp.dot(h_ref[...], w_ref[...],
                        preferred_element_type=jnp.float32)
            zbuf[pl.ds(nb * BN_BLK, BN_BLK), :] = z
            zm = _row_mask(nb, z)
            stats[0:1, :] += jnp.sum(zm, axis=0, keepdims=True)
            stats[1:2, :] += jnp.sum(zm * zm, axis=0, keepdims=True)

        @pl.when(p == 1)
        def _():
            a, bb = _ab(stats, g_ref[...], b_ref[...])
            zn = jnp.clip(a * zbuf[pl.ds(nb * BN_BLK, BN_BLK), :] + bb, 0.0, 6.0)
            for g in range(G):
                o_ref[g] = zn[:, g * Cg:(g + 1) * Cg]

    return pl.pallas_call(
        body,
        grid=(2, NB),
        in_specs=[
            pl.BlockSpec((BN_BLK, in_c), lambda p, nb: (nb, 0)),
            pl.BlockSpec((in_c, hid), lambda p, nb: (0, 0)),
            pl.BlockSpec((1, hid), lambda p, nb: (0, 0)),
            pl.BlockSpec((1, hid), lambda p, nb: (0, 0)),
        ],
        out_specs=pl.BlockSpec((G, BN_BLK, Cg), lambda p, nb: (0, nb, 0)),
        out_shape=jax.ShapeDtypeStruct((G, NPAD, Cg), jnp.float32),
        scratch_shapes=[
            pltpu.VMEM((NPAD, hid), jnp.float32),
            pltpu.VMEM((2, hid), jnp.float32),
        ],
    )


@functools.cache
def _bn_mm_bn_kernel(hid, Cg, out_c, use_res):
    G = hid // Cg

    def body(z_ref, w_ref, g2_ref, b2_ref, g3_ref, b3_ref, *rest):
        if use_res:
            id_ref, o_ref, zbuf, st2, st3 = rest
        else:
            o_ref, zbuf, st2, st3 = rest
        p = pl.program_id(0)
        nb = pl.program_id(1)

        @pl.when(jnp.logical_and(p == 0, nb == 0))
        def _():
            st2[...] = jnp.zeros_like(st2)
            st3[...] = jnp.zeros_like(st3)

        def z2blk():
            return jnp.concatenate([z_ref[g] for g in range(G)], axis=1)

        @pl.when(p == 0)
        def _():
            zm = _row_mask(nb, z2blk())
            st2[0:1, :] += jnp.sum(zm, axis=0, keepdims=True)
            st2[1:2, :] += jnp.sum(zm * zm, axis=0, keepdims=True)

        @pl.when(p == 1)
        def _():
            a, bb = _ab(st2, g2_ref[...], b2_ref[...])
            zn = jnp.clip(a * z2blk() + bb, 0.0, 6.0)
            z3 = jnp.dot(zn, w_ref[...], preferred_element_type=jnp.float32)
            zbuf[pl.ds(nb * BN_BLK, BN_BLK), :] = z3
            zm = _row_mask(nb, z3)
            st3[0:1, :] += jnp.sum(zm, axis=0, keepdims=True)
            st3[1:2, :] += jnp.sum(zm * zm, axis=0, keepdims=True)

        @pl.when(p == 2)
        def _():
            a, bb = _ab(st3, g3_ref[...], b3_ref[...])
            h = a * zbuf[pl.ds(nb * BN_BLK, BN_BLK), :] + bb
            if use_res:
                h = h + id_ref[...]
            o_ref[...] = _row_mask(nb, h)

    in_specs = [
        pl.BlockSpec((G, BN_BLK, Cg), lambda p, nb: (0, nb, 0)),
        pl.BlockSpec((hid, out_c), lambda p, nb: (0, 0)),
        pl.BlockSpec((1, hid), lambda p, nb: (0, 0)),
        pl.BlockSpec((1, hid), lambda p, nb: (0, 0)),
        pl.BlockSpec((1, out_c), lambda p, nb: (0, 0)),
        pl.BlockSpec((1, out_c), lambda p, nb: (0, 0)),
    ]
    if use_res:
        in_specs.append(pl.BlockSpec((BN_BLK, out_c), lambda p, nb: (nb, 0)))
    return pl.pallas_call(
        body,
        grid=(3, NB),
        in_specs=in_specs,
        out_specs=pl.BlockSpec((BN_BLK, out_c), lambda p, nb: (nb, 0)),
        out_shape=jax.ShapeDtypeStruct((NPAD, out_c), jnp.float32),
        scratch_shapes=[
            pltpu.VMEM((NPAD, out_c), jnp.float32),
            pltpu.VMEM((2, hid), jnp.float32),
            pltpu.VMEM((2, out_c), jnp.float32),
        ],
    )


@functools.cache
def _stem_bn_kernel():
    def body(z_ref, g_ref, b_ref, o_ref, zbuf, stats):
        p = pl.program_id(0)
        nb = pl.program_id(1)

        @pl.when(jnp.logical_and(p == 0, nb == 0))
        def _():
            stats[...] = jnp.zeros_like(stats)

        @pl.when(p == 0)
        def _():
            z = z_ref[0] + z_ref[1]
            zbuf[pl.ds(nb * BN_BLK, BN_BLK), :] = z
            zm = _row_mask(nb, z)
            stats[0:1, :] += jnp.sum(zm, axis=0, keepdims=True)
            stats[1:2, :] += jnp.sum(zm * zm, axis=0, keepdims=True)

        @pl.when(p == 1)
        def _():
            a, bb = _ab(stats, g_ref[...], b_ref[...])
            zn = jnp.clip(a * zbuf[pl.ds(nb * BN_BLK, BN_BLK), :] + bb, 0.0, 6.0)
            o_ref[...] = _row_mask(nb, zn)

    return pl.pallas_call(
        body,
        grid=(2, NB),
        in_specs=[
            pl.BlockSpec((2, BN_BLK, STEM), lambda p, nb: (0, nb, 0)),
            pl.BlockSpec((1, STEM), lambda p, nb: (0, 0)),
            pl.BlockSpec((1, STEM), lambda p, nb: (0, 0)),
        ],
        out_specs=pl.BlockSpec((BN_BLK, STEM), lambda p, nb: (nb, 0)),
        out_shape=jax.ShapeDtypeStruct((NPAD, STEM), jnp.float32),
        scratch_shapes=[
            pltpu.VMEM((NPAD, STEM), jnp.float32),
            pltpu.VMEM((2, STEM), jnp.float32),
        ],
    )


@functools.cache
def _head_kernel(in_c, fin, ncls):
    def body(h_ref, w8_ref, g_ref, b_ref, wfc_ref, bfc_ref, o_ref,
             stats, psum):
        p = pl.program_id(0)
        nb = pl.program_id(1)

        @pl.when(jnp.logical_and(p == 0, nb == 0))
        def _():
            stats[...] = jnp.zeros_like(stats)
            psum[...] = jnp.zeros_like(psum)

        @pl.when(p == 0)
        def _():
            z = jnp.dot(h_ref[...], w8_ref[...],
                        preferred_element_type=jnp.float32)
            zm = _row_mask(nb, z)
            stats[0:1, :] += jnp.sum(zm, axis=0, keepdims=True)
            stats[1:2, :] += jnp.sum(zm * zm, axis=0, keepdims=True)

        @pl.when(p == 1)
        def _():
            z = jnp.dot(h_ref[...], w8_ref[...],
                        preferred_element_type=jnp.float32)
            a, bb = _ab(stats, g_ref[...], b_ref[...])
            zn = jnp.clip(a * z + bb, 0.0, 6.0)
            psum[...] += jnp.sum(_row_mask(nb, zn), axis=0, keepdims=True)

        @pl.when(jnp.logical_and(p == 2, nb == 0))
        def _():
            pooled = psum[...] * (1.0 / N)
            o_ref[...] = (jnp.dot(pooled, wfc_ref[...],
                                  preferred_element_type=jnp.float32)
                          + bfc_ref[...])

    return pl.pallas_call(
        body,
        grid=(3, NB),
        in_specs=[
            pl.BlockSpec((BN_BLK, in_c), lambda p, nb: (nb, 0)),
            pl.BlockSpec((in_c, fin), lambda p, nb: (0, 0)),
            pl.BlockSpec((1, fin), lambda p, nb: (0, 0)),
            pl.BlockSpec((1, fin), lambda p, nb: (0, 0)),
            pl.BlockSpec((fin, ncls), lambda p, nb: (0, 0)),
            pl.BlockSpec((1, ncls), lambda p, nb: (0, 0)),
        ],
        out_specs=pl.BlockSpec((1, ncls), lambda p, nb: (0, 0)),
        out_shape=jax.ShapeDtypeStruct((1, ncls), jnp.float32),
        scratch_shapes=[
            pltpu.VMEM((2, fin), jnp.float32),
            pltpu.VMEM((1, fin), jnp.float32),
        ],
    )


def kernel(x, params, edge_index, kernel_offsets):
    meta = _edge_meta(edge_index, kernel_offsets)
    xpad = jnp.pad(x, ((0, NPAD - N), (0, 0)))
    z3stem = jnp.concatenate(
        [jnp.broadcast_to(xpad[:, 0:1], (NPAD, STEM)),
         jnp.broadcast_to(xpad[:, 1:2], (NPAD, STEM))], axis=0)
    wstem = jnp.concatenate(
        [params['Wc1'][:, 0, :], params['Wc1'][:, 1, :]], axis=1)
    hs = _dw_conv(z3stem, wstem, meta)
    h = _stem_bn_kernel()(hs.reshape(2, NPAD, STEM),
                          params['g0'][None], params['b0'][None])
    in_c = STEM
    bi = 0
    for t, c, nrep, s in SETTING:
        for i in range(nrep):
            stride = s if i == 0 else 1
            p = params['blocks'][bi]
            bi += 1
            use_res = (in_c == c and stride == 1)
            hid = in_c * t
            Cg = _CG[hid]
            z3 = _mm_bn_kernel(in_c, hid, Cg)(
                h, p['W1'], p['g1'][None], p['b1'][None])
            z2 = _dw_conv(z3.reshape(hid // Cg * NPAD, Cg), p['Wdw'], meta)
            args = (z2.reshape(hid // Cg, NPAD, Cg), p['W3'],
                    p['g2'][None], p['b2'][None], p['g3'][None], p['b3'][None])
            if use_res:
                args = args + (h,)
            h = _bn_mm_bn_kernel(hid, Cg, c, use_res)(*args)
            in_c = c
    out = _head_kernel(in_c, params['Wc8'].shape[1], params['bfc'].shape[0])(
        h, params['Wc8'], params['g8'][None], params['b8'][None],
        params['Wfc'], params['bfc'][None])
    return out
